# baseline (speedup 1.0000x reference)
import functools

import jax
import jax.numpy as jnp
from jax import lax
from jax.experimental import pallas as pl
from jax.experimental.pallas import tpu as pltpu
from jax.experimental.pallas import tpu_sc as plsc

_W = 16
_L = 16
_NC = 2
_NS = 16
_NW = _NC * _NS
_R = 16
_NBUF = 4


def _make_sc_add(rows, d):
    rpw = rows // _NW
    nchunk = rpw // _R
    mesh = plsc.VectorSubcoreMesh(core_axis_name="c", subcore_axis_name="s")

    @functools.partial(
        pl.kernel,
        mesh=mesh,
        out_type=jax.ShapeDtypeStruct((rows, d), jnp.float32),
        scratch_types=[
            pltpu.VMEM((_W, d), jnp.float32),
        ] + [pltpu.VMEM((_R, d), jnp.float32)] * _NBUF
          + [pltpu.SemaphoreType.DMA] * (2 * _NBUF),
    )
    def sc_add(x_hbm, t_hbm, o_hbm, tab, b0, b1, b2, b3,
               si0, si1, si2, si3, so0, so1, so2, so3):
        wid = lax.axis_index("s") * _NC + lax.axis_index("c")
        base = wid * rpw
        bufs = (b0, b1, b2, b3)
        sis = (si0, si1, si2, si3)
        sos = (so0, so1, so2, so3)

        def cin(g):
            b = g % _NBUF
            return pltpu.make_async_copy(
                x_hbm.at[pl.ds(base + g * _R, _R)], bufs[b], sis[b])

        def cout(g):
            b = g % _NBUF
            return pltpu.make_async_copy(
                bufs[b], o_hbm.at[pl.ds(base + g * _R, _R)], sos[b])

        pltpu.sync_copy(t_hbm, tab)

        def compute(b):
            buf = bufs[b]

            def jbody(j, c):
                s = pl.ds(j * _L, _L)
                for k in range(_R):
                    plsc.addupdate(buf.at[k, s], tab[k % _W, s])
                return c

            lax.fori_loop(0, d // _L, jbody, 0)

        def step(g, wait_out, start_in):
            b = g % _NBUF
            cin(g).wait()
            compute(b)
            cout(g).start()
            if wait_out:
                cout(g - 2).wait()
            if start_in:
                cin(g + 2).start()

        cin(0).start()
        cin(1).start()
        for g in (0, 1):
            step(g, wait_out=False, start_in=True)
        for g in (2, 3):
            step(g, wait_out=True, start_in=True)

        def gbody(i, c):
            g0 = 4 * i
            for b in range(_NBUF):
                g = g0 + b
                bb = b
                cin_d = pltpu.make_async_copy(
                    x_hbm.at[pl.ds(base + g * _R, _R)], bufs[bb], sis[bb])
                cin_d.wait()
                compute(bb)
                pltpu.make_async_copy(
                    bufs[bb], o_hbm.at[pl.ds(base + g * _R, _R)], sos[bb]).start()
                b2i = (b - 2) % _NBUF
                pltpu.make_async_copy(
                    bufs[b2i], o_hbm.at[pl.ds(base + (g - 2) * _R, _R)], sos[b2i]).wait()
                b3i = (b + 2) % _NBUF
                pltpu.make_async_copy(
                    x_hbm.at[pl.ds(base + (g + 2) * _R, _R)], bufs[b3i], sis[b3i]).start()
            return c

        lax.fori_loop(1, nchunk // 4 - 1, gbody, 0)

        for g in (nchunk - 4, nchunk - 3):
            step(g, wait_out=True, start_in=True)
        for g in (nchunk - 2, nchunk - 1):
            step(g, wait_out=True, start_in=False)
        for g in (nchunk - 2, nchunk - 1):
            cout(g).wait()

    return sc_add


def kernel(visn_feats, table):
    B, S, D = visn_feats.shape
    rows = B * S
    x2 = visn_feats.reshape(rows, D)
    out = _make_sc_add(rows, D)(x2, table)
    return out.reshape(B, S, D)


# speedup vs baseline: 2.1227x; 2.1227x over previous
import functools

import jax
import jax.numpy as jnp
from jax import lax
from jax.experimental import pallas as pl
from jax.experimental.pallas import tpu as pltpu
from jax.experimental.pallas import tpu_sc as plsc

_W = 16
_L = 16
_NC = 2
_NS = 16
_NW = _NC * _NS
_R = 32
_NBUF = 3


def _make_sc_add(rows, d):
    rpw = rows // _NW
    nchunk = rpw // _R
    mesh = plsc.VectorSubcoreMesh(core_axis_name="c", subcore_axis_name="s")

    @functools.partial(
        pl.kernel,
        mesh=mesh,
        out_type=jax.ShapeDtypeStruct((rows, d), jnp.float32),
        scratch_types=[
            pltpu.VMEM((_W, d), jnp.float32),
        ] + [pltpu.VMEM((_R, d), jnp.float32)] * _NBUF
          + [pltpu.SemaphoreType.DMA] * (2 * _NBUF),
    )
    def sc_add(x_hbm, t_hbm, o_hbm, tab, b0, b1, b2, si0, si1, si2, so0, so1, so2):
        wid = lax.axis_index("s") * _NC + lax.axis_index("c")
        base = wid * rpw
        bufs = (b0, b1, b2)
        sis = (si0, si1, si2)
        sos = (so0, so1, so2)

        pltpu.sync_copy(t_hbm, tab)

        def cin(g, b):
            return pltpu.make_async_copy(
                x_hbm.at[pl.ds(base + g * _R, _R)], bufs[b], sis[b])

        def cout(g, b):
            return pltpu.make_async_copy(
                bufs[b], o_hbm.at[pl.ds(base + g * _R, _R)], sos[b])

        def compute(b):
            buf = bufs[b]

            def jbody(j, c):
                s = pl.ds(j * _L, _L)
                ts = [tab[k, s] for k in range(_W)]
                for k in range(_R):
                    buf[k, s] = buf[k, s] + ts[k % _W]
                return c

            lax.fori_loop(0, d // _L, jbody, 0)

        def step(g, b, wait_out, start_in):
            cin(g, b).wait()
            if wait_out:
                cout(g - 2, (b + 1) % _NBUF).wait()
            if start_in:
                cin(g + 1, (g + 1) % _NBUF).start()
            compute(b)
            cout(g, b).start()

        cin(0, 0).start()
        step(0, 0, wait_out=False, start_in=True)
        step(1, 1, wait_out=False, start_in=True)
        step(2, 2, wait_out=True, start_in=True)

        def gbody(i, c):
            g0 = 3 * i
            for b in range(_NBUF):
                g = g0 + b
                cin(g, b).wait()
                cout(g - 2, (b + 1) % _NBUF).wait()
                cin(g + 1, (b + 1) % _NBUF).start()
                compute(b)
                cout(g, b).start()
            return c

        lax.fori_loop(1, nchunk // 3, gbody, 0)

        step(nchunk - 2, (nchunk - 2) % _NBUF, wait_out=True, start_in=True)
        step(nchunk - 1, (nchunk - 1) % _NBUF, wait_out=True, start_in=False)
        for g in (nchunk - 2, nchunk - 1):
            cout(g, g % _NBUF).wait()

    return sc_add


def kernel(visn_feats, table):
    B, S, D = visn_feats.shape
    rows = B * S
    x2 = visn_feats.reshape(rows, D)
    out = _make_sc_add(rows, D)(x2, table)
    return out.reshape(B, S, D)


# speedup vs baseline: 2.1410x; 1.0086x over previous
import functools

import jax
import jax.numpy as jnp
from jax import lax
from jax.experimental import pallas as pl
from jax.experimental.pallas import tpu as pltpu
from jax.experimental.pallas import tpu_sc as plsc

_W = 16
_L = 16
_NC = 2
_NS = 16
_NW = _NC * _NS
_R = 32
_NBUF = 3


def _make_sc_add(rows, d):
    rpw = rows // _NW
    nchunk = rpw // _R
    mesh = plsc.VectorSubcoreMesh(core_axis_name="c", subcore_axis_name="s")

    @functools.partial(
        pl.kernel,
        mesh=mesh,
        out_type=jax.ShapeDtypeStruct((rows, d), jnp.float32),
        scratch_types=[
            pltpu.VMEM((_W, d), jnp.float32),
        ] + [pltpu.VMEM((_R, d), jnp.float32)] * _NBUF
          + [pltpu.SemaphoreType.DMA] * (2 * _NBUF),
    )
    def sc_add(x_hbm, t_hbm, o_hbm, tab, b0, b1, b2, si0, si1, si2, so0, so1, so2):
        wid = lax.axis_index("s") * _NC + lax.axis_index("c")
        base = wid * rpw
        bufs = (b0, b1, b2)
        sis = (si0, si1, si2)
        sos = (so0, so1, so2)

        pltpu.sync_copy(t_hbm, tab)

        def cin(g, b):
            return pltpu.make_async_copy(
                x_hbm.at[pl.ds(base + g * _R, _R)], bufs[b], sis[b])

        def cout(g, b):
            return pltpu.make_async_copy(
                bufs[b], o_hbm.at[pl.ds(base + g * _R, _R)], sos[b])

        def compute(b):
            buf = bufs[b]

            @plsc.parallel_loop(0, d // _L, 1)
            def jbody(j):
                s = pl.ds(j * _L, _L)
                ts = [tab[k, s] for k in range(_W)]
                for k in range(_R):
                    buf[k, s] = buf[k, s] + ts[k % _W]

        def step(g, b, wait_out, start_in):
            cin(g, b).wait()
            if wait_out:
                cout(g - 2, (b + 1) % _NBUF).wait()
            if start_in:
                cin(g + 1, (g + 1) % _NBUF).start()
            compute(b)
            cout(g, b).start()

        cin(0, 0).start()
        step(0, 0, wait_out=False, start_in=True)
        step(1, 1, wait_out=False, start_in=True)
        step(2, 2, wait_out=True, start_in=True)

        def gbody(i, c):
            g0 = 3 * i
            for b in range(_NBUF):
                g = g0 + b
                cin(g, b).wait()
                cout(g - 2, (b + 1) % _NBUF).wait()
                cin(g + 1, (b + 1) % _NBUF).start()
                compute(b)
                cout(g, b).start()
            return c

        lax.fori_loop(1, nchunk // 3, gbody, 0)

        step(nchunk - 2, (nchunk - 2) % _NBUF, wait_out=True, start_in=True)
        step(nchunk - 1, (nchunk - 1) % _NBUF, wait_out=True, start_in=False)
        for g in (nchunk - 2, nchunk - 1):
            cout(g, g % _NBUF).wait()

    return sc_add


def kernel(visn_feats, table):
    B, S, D = visn_feats.shape
    rows = B * S
    x2 = visn_feats.reshape(rows, D)
    out = _make_sc_add(rows, D)(x2, table)
    return out.reshape(B, S, D)
